# baseline (speedup 1.0000x reference)
import functools

import jax
import jax.numpy as jnp
from jax import lax
from jax.experimental import pallas as pl
from jax.experimental.pallas import tpu as pltpu
from jax.experimental.pallas import tpu_sc as plsc

_K = 8192
_D = 32
_NPAD = 800
_NW = 32
_RPW = 25
_BK = 1024
_NB = _K // _BK
_NCAND = 2 * _NB
_BIG = 2**31 - 1


def _cand_kernel(z2_ref, cb_ref, i1_ref, i2_ref, cbp_ref):
    k_blk = pl.program_id(0)
    z2 = z2_ref[...]
    cb = cb_ref[...]

    cbp_ref[:, : _D] = cb

    cn = jnp.sum(cb * cb, axis=1)
    cb_aug = jnp.concatenate([cb, cn[:, None]], axis=1)
    s = lax.dot_general(
        cb_aug, z2, (((1,), (1,)), ((), ())),
        preferred_element_type=jnp.float32,
        precision=lax.Precision.HIGHEST)

    idx = lax.broadcasted_iota(jnp.int32, s.shape, 0) + k_blk * _BK
    m1 = jnp.min(s, axis=0)
    i1 = jnp.min(jnp.where(s == m1[None, :], idx, jnp.int32(_BIG)), axis=0)
    s2 = jnp.where(idx == i1[None, :], jnp.float32(jnp.inf), s)
    m2 = jnp.min(s2, axis=0)
    i2 = jnp.min(jnp.where(s2 == m2[None, :], idx, jnp.int32(_BIG)), axis=0)

    i1_ref[...] = i1[None, None, :]
    i2_ref[...] = i2[None, None, :]


def _candidates(z2, cb):
    return pl.pallas_call(
        _cand_kernel,
        grid=(_NB,),
        in_specs=[
            pl.BlockSpec((_NPAD, _D + 1), lambda k: (0, 0)),
            pl.BlockSpec((_BK, _D), lambda k: (k, 0)),
        ],
        out_specs=[
            pl.BlockSpec((1, 1, _NPAD), lambda k: (k, 0, 0)),
            pl.BlockSpec((1, 1, _NPAD), lambda k: (k, 0, 0)),
            pl.BlockSpec((_BK, 128), lambda k: (k, 0)),
        ],
        out_shape=[
            jax.ShapeDtypeStruct((_NB, 1, _NPAD), jnp.int32),
            jax.ShapeDtypeStruct((_NB, 1, _NPAD), jnp.int32),
            jax.ShapeDtypeStruct((_K, 128), jnp.float32),
        ],
    )(z2, cb)


_GDN = lax.GatherDimensionNumbers(
    offset_dims=(), collapsed_slice_dims=(0,), start_index_map=(0,))


def _lperm(v, idx):
    return lax.gather(v, idx[:, None], _GDN, (1,),
                      mode=lax.GatherScatterMode.PROMISE_IN_BOUNDS)


def _sc_rescue_kernel(zw_hbm, cand_hbm, cbp_hbm, out_hbm,
                      zbuf, candbuf, crows, ansbuf, sem_in, sem_g):
    cid = lax.axis_index("c")
    sid = lax.axis_index("s")
    wid = sid * 2 + cid

    cp_z = pltpu.async_copy(zw_hbm.at[wid], zbuf, sem_in)
    cp_c = pltpu.async_copy(cand_hbm.at[wid], candbuf, sem_in)
    cp_z.wait()
    cp_c.wait()

    gathers = []
    for g in range(5):
        gathers.append(pltpu.async_copy(
            cbp_hbm.at[candbuf.at[pl.ds(g * 80, 80)]],
            crows.at[pl.ds(g * 80, 80)],
            sem_g))
    for g in gathers:
        g.wait()

    lanes = lax.iota(jnp.int32, 16)
    zeros = lanes * 0

    def row_body(r, acc):
        acc0, acc1 = acc
        z0 = zbuf[pl.ds(r * _D, 16)]
        z1 = zbuf[pl.ds(r * _D + 16, 16)]

        val = jnp.full((16,), jnp.inf, jnp.float32)
        for j in range(_NCAND):
            row = r * _NCAND + j
            c0 = crows[row, pl.ds(0, 16)]
            c1 = crows[row, pl.ds(16, 16)]
            t0 = z0 - c0
            t1 = z1 - c1
            b = t0 * t0 + t1 * t1
            for st in (8, 4, 2, 1):
                b = b + _lperm(b, (lanes + st) & 15)
            bs = _lperm(b, zeros)
            val = jnp.where(lanes == j, bs, val)

        bi = candbuf[pl.ds(r * _NCAND, _NCAND)]
        bv = val
        for st in (8, 4, 2, 1):
            perm = (lanes + st) & 15
            v2 = _lperm(bv, perm)
            i2v = _lperm(bi, perm)
            t2 = (v2 < bv) | ((v2 == bv) & (i2v < bi))
            bv = jnp.where(t2, v2, bv)
            bi = jnp.where(t2, i2v, bi)
        ans = _lperm(bi, zeros)

        acc0 = jnp.where(lanes == r, ans, acc0)
        acc1 = jnp.where(lanes == (r - 16), ans, acc1)
        return acc0, acc1

    zero = jnp.zeros((16,), jnp.int32)
    acc0, acc1 = lax.fori_loop(0, _RPW, row_body, (zero, zero))
    ansbuf[pl.ds(0, 16)] = acc0
    ansbuf[pl.ds(16, 16)] = acc1
    pltpu.sync_copy(ansbuf, out_hbm.at[wid])


def _sc_rescue(zw, candw, cbp):
    mesh = plsc.VectorSubcoreMesh(core_axis_name="c", subcore_axis_name="s")
    kern = functools.partial(
        pl.kernel,
        out_type=jax.ShapeDtypeStruct((_NW, 32), jnp.int32),
        mesh=mesh,
        scratch_types=[
            pltpu.VMEM((_RPW * _D,), jnp.float32),
            pltpu.VMEM((_RPW * _NCAND,), jnp.int32),
            pltpu.VMEM((_RPW * _NCAND, 128), jnp.float32),
            pltpu.VMEM((32,), jnp.int32),
            pltpu.SemaphoreType.DMA,
            pltpu.SemaphoreType.DMA,
        ],
    )(_sc_rescue_kernel)
    return kern(zw, candw, cbp)


def kernel(inputs, codebook):
    b, t, d = inputs.shape
    z = inputs.reshape(b * t, d)
    z = jnp.pad(z, ((0, _NPAD - b * t), (0, 0)))

    z2 = jnp.concatenate(
        [z * jnp.float32(-2), jnp.ones((_NPAD, 1), jnp.float32)], axis=1)
    i1, i2, cbp = _candidates(z2, codebook)

    cands = jnp.concatenate([i1[:, 0, :].T, i2[:, 0, :].T], axis=1)
    candw = cands.reshape(_NW, _RPW * _NCAND)
    zw = z.reshape(_NW, _RPW * _D)

    out = _sc_rescue(zw, candw, cbp)
    idx = out[:, :_RPW].reshape(_NW * _RPW)[: b * t]
    return idx.reshape(b, t)


# speedup vs baseline: 1.1428x; 1.1428x over previous
import functools

import jax
import jax.numpy as jnp
from jax import lax
from jax.experimental import pallas as pl
from jax.experimental.pallas import tpu as pltpu
from jax.experimental.pallas import tpu_sc as plsc

_K = 8192
_D = 32
_NPAD = 800
_NW = 32
_RPW = 25
_BK = 1024
_NB = _K // _BK
_NCAND = 2 * _NB
_BIG = 2**31 - 1


def _cand_kernel(z2t_ref, cb_ref, i1_ref, i2_ref, cbp_ref):
    k_blk = pl.program_id(0)
    z2t = z2t_ref[...]
    cb = cb_ref[...]

    cbp_ref[:, : _D] = cb

    cn = jnp.sum(cb * cb, axis=1)
    cb_aug = jnp.concatenate([cb, cn[:, None]], axis=1)
    s = lax.dot_general(
        cb_aug, z2t, (((1,), (0,)), ((), ())),
        preferred_element_type=jnp.float32,
        precision=lax.Precision.HIGHEST)

    def lexlt(va, ia, vb, ib):
        return (va < vb) | ((va == vb) & (ia < ib))

    def merge(A, B):
        a1, ai1, a2, ai2 = A
        b1, bi1, b2, bi2 = B
        bwin = lexlt(b1, bi1, a1, ai1)
        n1 = jnp.where(bwin, b1, a1)
        ni1 = jnp.where(bwin, bi1, ai1)
        l1 = jnp.where(bwin, a1, b1)
        li1 = jnp.where(bwin, ai1, bi1)
        w2 = jnp.where(bwin, b2, a2)
        wi2 = jnp.where(bwin, bi2, ai2)
        s2w = lexlt(w2, wi2, l1, li1)
        return (n1, ni1,
                jnp.where(s2w, w2, l1), jnp.where(s2w, wi2, li1))

    idx8 = lax.broadcasted_iota(jnp.int32, (8, _NPAD), 0) + k_blk * _BK
    inf8 = jnp.full((8, _NPAD), jnp.inf, jnp.float32)
    zero8 = jnp.zeros((8, _NPAD), jnp.int32)
    accs = []
    for a in range(4):
        m1, i1, m2, i2 = inf8, zero8, inf8, zero8
        for m in range(_BK // 32):
            v = m * 4 + a
            sv = lax.slice(s, (8 * v, 0), (8 * v + 8, _NPAD))
            iv = idx8 + 8 * v
            c1 = sv < m1
            c2 = sv < m2
            m2 = jnp.where(c1, m1, jnp.where(c2, sv, m2))
            i2 = jnp.where(c1, i1, jnp.where(c2, iv, i2))
            m1 = jnp.where(c1, sv, m1)
            i1 = jnp.where(c1, iv, i1)
        accs.append((m1, i1, m2, i2))

    t = merge(merge(accs[0], accs[1]), merge(accs[2], accs[3]))
    r = tuple(lax.slice(x, (0, 0), (1, _NPAD)) for x in t)
    for u in range(1, 8):
        r = merge(r, tuple(lax.slice(x, (u, 0), (u + 1, _NPAD)) for x in t))

    i1_ref[...] = r[1][None]
    i2_ref[...] = r[3][None]


def _candidates(z2t, cb):
    return pl.pallas_call(
        _cand_kernel,
        grid=(_NB,),
        in_specs=[
            pl.BlockSpec((_D + 1, _NPAD), lambda k: (0, 0)),
            pl.BlockSpec((_BK, _D), lambda k: (k, 0)),
        ],
        out_specs=[
            pl.BlockSpec((1, 1, _NPAD), lambda k: (k, 0, 0)),
            pl.BlockSpec((1, 1, _NPAD), lambda k: (k, 0, 0)),
            pl.BlockSpec((_BK, 128), lambda k: (k, 0)),
        ],
        out_shape=[
            jax.ShapeDtypeStruct((_NB, 1, _NPAD), jnp.int32),
            jax.ShapeDtypeStruct((_NB, 1, _NPAD), jnp.int32),
            jax.ShapeDtypeStruct((_K, 128), jnp.float32),
        ],
    )(z2t, cb)


_GDN = lax.GatherDimensionNumbers(
    offset_dims=(), collapsed_slice_dims=(0,), start_index_map=(0,))


def _lperm(v, idx):
    return lax.gather(v, idx[:, None], _GDN, (1,),
                      mode=lax.GatherScatterMode.PROMISE_IN_BOUNDS)


def _sc_rescue_kernel(zw_hbm, cand_hbm, cbp_hbm, out_hbm,
                      zbuf, candbuf, crows, ansbuf, sem_in, sem_g):
    cid = lax.axis_index("c")
    sid = lax.axis_index("s")
    wid = sid * 2 + cid

    cp_z = pltpu.async_copy(zw_hbm.at[wid], zbuf, sem_in)
    cp_c = pltpu.async_copy(cand_hbm.at[wid], candbuf, sem_in)
    cp_z.wait()
    cp_c.wait()

    gathers = []
    for g in range(5):
        gathers.append(pltpu.async_copy(
            cbp_hbm.at[candbuf.at[pl.ds(g * 80, 80)]],
            crows.at[pl.ds(g * 80, 80)],
            sem_g))
    for g in gathers:
        g.wait()

    lanes = lax.iota(jnp.int32, 16)
    zeros = lanes * 0

    def row_body(r, acc):
        acc0, acc1 = acc
        z0 = zbuf[pl.ds(r * _D, 16)]
        z1 = zbuf[pl.ds(r * _D + 16, 16)]

        val = jnp.full((16,), jnp.inf, jnp.float32)
        for j in range(_NCAND):
            row = r * _NCAND + j
            c0 = crows[row, pl.ds(0, 16)]
            c1 = crows[row, pl.ds(16, 16)]
            t0 = z0 - c0
            t1 = z1 - c1
            b = t0 * t0 + t1 * t1
            for st in (8, 4, 2, 1):
                b = b + _lperm(b, (lanes + st) & 15)
            bs = _lperm(b, zeros)
            val = jnp.where(lanes == j, bs, val)

        bi = candbuf[pl.ds(r * _NCAND, _NCAND)]
        bv = val
        for st in (8, 4, 2, 1):
            perm = (lanes + st) & 15
            v2 = _lperm(bv, perm)
            i2v = _lperm(bi, perm)
            t2 = (v2 < bv) | ((v2 == bv) & (i2v < bi))
            bv = jnp.where(t2, v2, bv)
            bi = jnp.where(t2, i2v, bi)
        ans = _lperm(bi, zeros)

        acc0 = jnp.where(lanes == r, ans, acc0)
        acc1 = jnp.where(lanes == (r - 16), ans, acc1)
        return acc0, acc1

    zero = jnp.zeros((16,), jnp.int32)
    acc0, acc1 = lax.fori_loop(0, _RPW, row_body, (zero, zero))
    ansbuf[pl.ds(0, 16)] = acc0
    ansbuf[pl.ds(16, 16)] = acc1
    pltpu.sync_copy(ansbuf, out_hbm.at[wid])


def _sc_rescue(zw, candw, cbp):
    mesh = plsc.VectorSubcoreMesh(core_axis_name="c", subcore_axis_name="s")
    kern = functools.partial(
        pl.kernel,
        out_type=jax.ShapeDtypeStruct((_NW, 32), jnp.int32),
        mesh=mesh,
        scratch_types=[
            pltpu.VMEM((_RPW * _D,), jnp.float32),
            pltpu.VMEM((_RPW * _NCAND,), jnp.int32),
            pltpu.VMEM((_RPW * _NCAND, 128), jnp.float32),
            pltpu.VMEM((32,), jnp.int32),
            pltpu.SemaphoreType.DMA,
            pltpu.SemaphoreType.DMA,
        ],
    )(_sc_rescue_kernel)
    return kern(zw, candw, cbp)


def kernel(inputs, codebook):
    b, t, d = inputs.shape
    z = inputs.reshape(b * t, d)
    z = jnp.pad(z, ((0, _NPAD - b * t), (0, 0)))

    z2t = jnp.concatenate(
        [z.T * jnp.float32(-2), jnp.ones((1, _NPAD), jnp.float32)], axis=0)
    i1, i2, cbp = _candidates(z2t, codebook)

    cands = jnp.concatenate([i1[:, 0, :].T, i2[:, 0, :].T], axis=1)
    candw = cands.reshape(_NW, _RPW * _NCAND)
    zw = z.reshape(_NW, _RPW * _D)

    out = _sc_rescue(zw, candw, cbp)
    idx = out[:, :_RPW].reshape(_NW * _RPW)[: b * t]
    return idx.reshape(b, t)
